# baseline (speedup 1.0000x reference)
import functools

import jax
import jax.numpy as jnp
from jax import lax
from jax.experimental import pallas as pl
from jax.experimental.pallas import tpu as pltpu
from jax.experimental.pallas import tpu_sc as plsc

_DIM = 128
_HALF = _DIM // 2
_BATCH = 16384
_NUM_OPS_GRP = 12500
_NC = 2
_NS = 16
_NW = _NC * _NS
_ROWS_PER_W = _BATCH // _NW
_PASS = 64
_NPASS = _ROWS_PER_W // _PASS
_NG = 3
_LANES = 16

_mesh = plsc.VectorSubcoreMesh(core_axis_name="c", subcore_axis_name="s")


@functools.partial(
    pl.kernel,
    out_type=jax.ShapeDtypeStruct((_BATCH, _DIM), jnp.float32),
    mesh=_mesh,
    scratch_types=(
        [pltpu.VMEM((_ROWS_PER_W + _LANES,), jnp.int32)]
        + [pltpu.VMEM((_PASS, _DIM), jnp.float32) for _ in range(_NPASS)]
        + [pltpu.VMEM((_PASS // 8, 8, _HALF), jnp.float32)
           for _ in range(2 * _NG)]
        + [pltpu.SemaphoreType.DMA for _ in range(_NG + _NPASS)]
    ),
)
def _sc_complex_diag(emb_hbm, idx_hbm, real_hbm, imag_hbm, out_hbm,
                     idx_v, *bufs):
    embs = bufs[:_NPASS]
    rbs = bufs[_NPASS:_NPASS + _NG]
    ibs = bufs[_NPASS + _NG:_NPASS + 2 * _NG]
    gsems = bufs[_NPASS + 2 * _NG:_NPASS + 3 * _NG]
    osems = bufs[_NPASS + 3 * _NG:]

    wid = lax.axis_index("s") * _NC + lax.axis_index("c")
    base = wid * _ROWS_PER_W

    pltpu.sync_copy(idx_hbm.at[pl.ds(base, _ROWS_PER_W)],
                    idx_v.at[pl.ds(0, _ROWS_PER_W)])

    def fire(p):
        gs = p % _NG
        row0 = base + p * _PASS
        pltpu.async_copy(emb_hbm.at[pl.ds(row0, _PASS)], embs[p], gsems[gs])

        @pl.loop(0, _PASS // _LANES)
        def _(t):
            v = idx_v[pl.ds(p * _PASS + t * _LANES, _LANES)]
            for j in range(_LANES):
                k = v[j]
                g = k >> 3
                s = k & 7
                r = t * _LANES + j
                pltpu.async_copy(real_hbm.at[g, s],
                                 rbs[gs].at[r >> 3, r & 7], gsems[gs])
                pltpu.async_copy(imag_hbm.at[g, s],
                                 ibs[gs].at[r >> 3, r & 7], gsems[gs])

    def drain(p):
        gs = p % _NG
        dummy = real_hbm.at[pl.ds(0, _PASS // 8)]
        pltpu.make_async_copy(dummy, rbs[gs], gsems[gs]).wait()
        pltpu.make_async_copy(dummy, ibs[gs], gsems[gs]).wait()
        pltpu.make_async_copy(emb_hbm.at[pl.ds(0, _PASS)], embs[p],
                              gsems[gs]).wait()

    def compute_and_write(p):
        gs = p % _NG
        row0 = base + p * _PASS
        emb_v, rb_v, ib_v = embs[p], rbs[gs], ibs[gs]

        @pl.loop(0, _PASS)
        def _(r):
            for c in range(_HALF // _LANES):
                lo = pl.ds(c * _LANES, _LANES)
                hi = pl.ds(_HALF + c * _LANES, _LANES)
                ra = emb_v[r, lo]
                ia = emb_v[r, hi]
                rb = rb_v[r >> 3, r & 7, lo]
                ib = ib_v[r >> 3, r & 7, lo]
                emb_v[r, lo] = ra * rb - ia * ib
                emb_v[r, hi] = ra * ib + ia * rb

        pltpu.async_copy(emb_v, out_hbm.at[pl.ds(row0, _PASS)], osems[p])

    fire(0)
    fire(1)
    for p in range(_NPASS):
        if p + 2 < _NPASS:
            fire(p + 2)
        drain(p)
        compute_and_write(p)
    for p in range(_NPASS):
        pltpu.make_async_copy(emb_hbm.at[pl.ds(0, _PASS)], embs[p],
                              osems[p]).wait()


def kernel(embeddings, operator_idxs, real, imag):
    idx = operator_idxs.astype(jnp.int32)
    real3 = real.reshape(_NUM_OPS_GRP, 8, _HALF)
    imag3 = imag.reshape(_NUM_OPS_GRP, 8, _HALF)
    return _sc_complex_diag(embeddings, idx, real3, imag3)


# speedup vs baseline: 1.0109x; 1.0109x over previous
import functools

import jax
import jax.numpy as jnp
from jax import lax
from jax.experimental import pallas as pl
from jax.experimental.pallas import tpu as pltpu
from jax.experimental.pallas import tpu_sc as plsc

_DIM = 128
_HALF = _DIM // 2
_BATCH = 16384
_NUM_OPS_GRP = 12500
_NC = 2
_NS = 16
_NW = _NC * _NS
_ROWS_PER_W = _BATCH // _NW
_PASS = 128
_NPASS = _ROWS_PER_W // _PASS
_LANES = 16

_mesh = plsc.VectorSubcoreMesh(core_axis_name="c", subcore_axis_name="s")


@functools.partial(
    pl.kernel,
    out_type=jax.ShapeDtypeStruct((_BATCH, _DIM), jnp.float32),
    mesh=_mesh,
    scratch_types=[
        pltpu.VMEM((_ROWS_PER_W + _LANES,), jnp.int32),
        pltpu.VMEM((_PASS, _DIM), jnp.float32),
        pltpu.VMEM((_PASS, _DIM), jnp.float32),
        pltpu.VMEM((_PASS, _DIM), jnp.float32),
        pltpu.VMEM((_PASS // 8, 8, _HALF), jnp.float32),
        pltpu.VMEM((_PASS // 8, 8, _HALF), jnp.float32),
        pltpu.VMEM((_PASS // 8, 8, _HALF), jnp.float32),
        pltpu.VMEM((_PASS // 8, 8, _HALF), jnp.float32),
        pltpu.SemaphoreType.DMA,
        pltpu.SemaphoreType.DMA,
        pltpu.SemaphoreType.DMA,
        pltpu.SemaphoreType.DMA,
        pltpu.SemaphoreType.DMA,
    ],
)
def _sc_complex_diag(emb_hbm, idx_hbm, real_hbm, imag_hbm, out_hbm,
                     idx_v, emb0_v, emb1_v, emb2_v, rb0_v, rb1_v, ib0_v,
                     ib1_v, gsem0, gsem1, osem0, osem1, osem2):
    wid = lax.axis_index("s") * _NC + lax.axis_index("c")
    base = wid * _ROWS_PER_W
    embs = (emb0_v, emb1_v, emb2_v)
    rbs = (rb0_v, rb1_v)
    ibs = (ib0_v, ib1_v)
    gsems = (gsem0, gsem1)
    osems = (osem0, osem1, osem2)

    pltpu.sync_copy(idx_hbm.at[pl.ds(base, _ROWS_PER_W)],
                    idx_v.at[pl.ds(0, _ROWS_PER_W)])

    def fire(p):
        g2, e3 = p & 1, p % 3
        row0 = base + p * _PASS
        pltpu.async_copy(emb_hbm.at[pl.ds(row0, _PASS)], embs[e3], gsems[g2])

        @pl.loop(0, _PASS // _LANES)
        def _(t):
            v = idx_v[pl.ds(p * _PASS + t * _LANES, _LANES)]
            for j in range(_LANES):
                k = v[j]
                g = k >> 3
                s = k & 7
                r = t * _LANES + j
                pltpu.async_copy(real_hbm.at[g, s],
                                 rbs[g2].at[r >> 3, r & 7], gsems[g2])
                pltpu.async_copy(imag_hbm.at[g, s],
                                 ibs[g2].at[r >> 3, r & 7], gsems[g2])

    def drain(p):
        g2 = p & 1
        dummy = real_hbm.at[pl.ds(0, _PASS // 8)]
        pltpu.make_async_copy(dummy, rbs[g2], gsems[g2]).wait()
        pltpu.make_async_copy(dummy, ibs[g2], gsems[g2]).wait()
        pltpu.make_async_copy(emb_hbm.at[pl.ds(0, _PASS)], embs[p % 3],
                              gsems[g2]).wait()

    def compute_and_write(p):
        g2, e3 = p & 1, p % 3
        row0 = base + p * _PASS
        emb_v, rb_v, ib_v = embs[e3], rbs[g2], ibs[g2]

        @pl.loop(0, _PASS)
        def _(r):
            for c in range(_HALF // _LANES):
                lo = pl.ds(c * _LANES, _LANES)
                hi = pl.ds(_HALF + c * _LANES, _LANES)
                ra = emb_v[r, lo]
                ia = emb_v[r, hi]
                rb = rb_v[r >> 3, r & 7, lo]
                ib = ib_v[r >> 3, r & 7, lo]
                emb_v[r, lo] = ra * rb - ia * ib
                emb_v[r, hi] = ra * ib + ia * rb

        pltpu.async_copy(emb_v, out_hbm.at[pl.ds(row0, _PASS)], osems[e3])

    def wait_out(p):
        pltpu.make_async_copy(emb_hbm.at[pl.ds(0, _PASS)], embs[p % 3],
                              osems[p % 3]).wait()

    fire(0)
    for p in range(_NPASS):
        if p + 1 < _NPASS:
            if p >= 2:
                wait_out(p - 2)
            fire(p + 1)
        drain(p)
        compute_and_write(p)
    wait_out(_NPASS - 2)
    wait_out(_NPASS - 1)


def kernel(embeddings, operator_idxs, real, imag):
    idx = operator_idxs.astype(jnp.int32)
    real3 = real.reshape(_NUM_OPS_GRP, 8, _HALF)
    imag3 = imag.reshape(_NUM_OPS_GRP, 8, _HALF)
    return _sc_complex_diag(embeddings, idx, real3, imag3)
